# initial kernel scaffold (unmeasured)
import jax
import jax.numpy as jnp
from jax import lax
from jax.experimental import pallas as pl
from jax.experimental.pallas import tpu as pltpu

N_DEV = 4
N_LAYERS = 3


def kernel(x, Win0, Wout0, Win1, Wout1, Win2, Wout2):
    m_per, d = x.shape
    M = N_DEV * m_per

    def body(x_ref, win0_ref, wout0_ref, win1_ref, wout1_ref, win2_ref,
             wout2_ref, out_ref, X_ref, acc_ref, circ_ref,
             ag_send, ag_recv, ar_send, ar_recv):
        my = lax.axis_index("i")
        left = (my - 1) % N_DEV
        right = (my + 1) % N_DEV

        barrier = pltpu.get_barrier_semaphore()
        for nbr in (left, right):
            pl.semaphore_signal(
                barrier, inc=1,
                device_id=(nbr,), device_id_type=pl.DeviceIdType.MESH,
            )
        pl.semaphore_wait(barrier, 2)

        X_ref[pl.ds(my * m_per, m_per), :] = x_ref[...].astype(jnp.bfloat16)
        for h in range(N_DEV - 1):
            origin = (my - h) % N_DEV
            sl = pl.ds(origin * m_per, m_per)
            rdma = pltpu.make_async_remote_copy(
                src_ref=X_ref.at[sl, :],
                dst_ref=X_ref.at[sl, :],
                send_sem=ag_send.at[h],
                recv_sem=ag_recv.at[h],
                device_id=(right,),
                device_id_type=pl.DeviceIdType.MESH,
            )
            rdma.start()
            rdma.wait()

        wins = (win0_ref, win1_ref, win2_ref)
        wouts = (wout0_ref, wout1_ref, wout2_ref)
        for l in range(N_LAYERS):
            Xb = X_ref[...]
            Wi = wins[l][...].astype(jnp.bfloat16)
            H = jnp.maximum(
                jnp.dot(Xb, Wi, preferred_element_type=jnp.float32), 0.0)
            Wo = wouts[l][...].astype(jnp.bfloat16)
            P = jnp.dot(H.astype(jnp.bfloat16), Wo,
                        preferred_element_type=jnp.float32)
            acc_ref[...] = P
            circ_ref[l, 0] = P.astype(jnp.bfloat16)

            for h in range(N_DEV - 1):
                rdma = pltpu.make_async_remote_copy(
                    src_ref=circ_ref.at[l, h],
                    dst_ref=circ_ref.at[l, h + 1],
                    send_sem=ar_send.at[l, h],
                    recv_sem=ar_recv.at[l, h],
                    device_id=(right,),
                    device_id_type=pl.DeviceIdType.MESH,
                )
                rdma.start()
                rdma.wait()
                acc_ref[...] += circ_ref[l, h + 1].astype(jnp.float32)

            if l < N_LAYERS - 1:
                X_ref[...] = acc_ref[...].astype(jnp.bfloat16)
            else:
                out_ref[...] = acc_ref[...]

    return pl.pallas_call(
        body,
        out_shape=jax.ShapeDtypeStruct((M, d), jnp.float32),
        in_specs=[pl.BlockSpec(memory_space=pltpu.VMEM)] * 7,
        out_specs=pl.BlockSpec(memory_space=pltpu.VMEM),
        scratch_shapes=[
            pltpu.VMEM((M, d), jnp.bfloat16),
            pltpu.VMEM((M, d), jnp.float32),
            pltpu.VMEM((N_LAYERS, N_DEV, M, d), jnp.bfloat16),
            pltpu.SemaphoreType.DMA((N_DEV - 1,)),
            pltpu.SemaphoreType.DMA((N_DEV - 1,)),
            pltpu.SemaphoreType.DMA((N_LAYERS, N_DEV - 1)),
            pltpu.SemaphoreType.DMA((N_LAYERS, N_DEV - 1)),
        ],
        compiler_params=pltpu.CompilerParams(collective_id=0),
    )(x, Win0, Wout0, Win1, Wout1, Win2, Wout2)


# baseline (device time: 106165 ns/iter reference)
import jax
import jax.numpy as jnp
from jax import lax
from jax.experimental import pallas as pl
from jax.experimental.pallas import tpu as pltpu

N_DEV = 4
N_LAYERS = 3


def kernel(x, Win0, Wout0, Win1, Wout1, Win2, Wout2):
    m_per, d = x.shape
    M = N_DEV * m_per

    def body(x_ref, win0_ref, wout0_ref, win1_ref, wout1_ref, win2_ref,
             wout2_ref, out_ref, X_ref, acc_ref, circ_ref,
             ag_send, ag_recv, ar_send, ar_recv):
        my = lax.axis_index("i")
        left = (my - 1) % N_DEV
        right = (my + 1) % N_DEV

        barrier = pltpu.get_barrier_semaphore()
        for nbr in (left, right):
            pl.semaphore_signal(
                barrier, inc=1,
                device_id=(nbr,), device_id_type=pl.DeviceIdType.MESH,
            )
        pl.semaphore_wait(barrier, 2)

        X_ref[pl.ds(my * m_per, m_per), :] = x_ref[...].astype(jnp.bfloat16)
        for h in range(N_DEV - 1):
            origin = (my - h) % N_DEV
            sl = pl.ds(origin * m_per, m_per)
            rdma = pltpu.make_async_remote_copy(
                src_ref=X_ref.at[sl, :],
                dst_ref=X_ref.at[sl, :],
                send_sem=ag_send.at[h],
                recv_sem=ag_recv.at[h],
                device_id=(right,),
                device_id_type=pl.DeviceIdType.MESH,
            )
            rdma.start()
            rdma.wait()

        wins = (win0_ref, win1_ref, win2_ref)
        wouts = (wout0_ref, wout1_ref, wout2_ref)
        for l in range(N_LAYERS):
            Xb = X_ref[...]
            Wi = wins[l][...].astype(jnp.bfloat16)
            H = jnp.maximum(
                jnp.dot(Xb, Wi, preferred_element_type=jnp.float32), 0.0)
            Wo = wouts[l][...].astype(jnp.bfloat16)
            P = jnp.dot(H.astype(jnp.bfloat16), Wo,
                        preferred_element_type=jnp.float32)
            acc_ref[...] = P
            circ_ref[l, 0] = P.astype(jnp.bfloat16)

            for h in range(N_DEV - 1):
                rdma = pltpu.make_async_remote_copy(
                    src_ref=circ_ref.at[l, h],
                    dst_ref=circ_ref.at[l, h + 1],
                    send_sem=ar_send.at[l, h],
                    recv_sem=ar_recv.at[l, h],
                    device_id=(right,),
                    device_id_type=pl.DeviceIdType.MESH,
                )
                rdma.start()
                rdma.wait()
                acc_ref[...] += circ_ref[l, h + 1].astype(jnp.float32)

            if l < N_LAYERS - 1:
                X_ref[...] = acc_ref[...].astype(jnp.bfloat16)
            else:
                out_ref[...] = acc_ref[...]

    return pl.pallas_call(
        body,
        out_shape=jax.ShapeDtypeStruct((M, d), jnp.float32),
        in_specs=[pl.BlockSpec(memory_space=pltpu.VMEM)] * 7,
        out_specs=pl.BlockSpec(memory_space=pltpu.VMEM),
        scratch_shapes=[
            pltpu.VMEM((M, d), jnp.bfloat16),
            pltpu.VMEM((M, d), jnp.float32),
            pltpu.VMEM((N_LAYERS, N_DEV, M, d), jnp.bfloat16),
            pltpu.SemaphoreType.DMA((N_DEV - 1,)),
            pltpu.SemaphoreType.DMA((N_DEV - 1,)),
            pltpu.SemaphoreType.DMA((N_LAYERS, N_DEV - 1)),
            pltpu.SemaphoreType.DMA((N_LAYERS, N_DEV - 1)),
        ],
        compiler_params=pltpu.CompilerParams(
            collective_id=0, vmem_limit_bytes=100 * 1024 * 1024),
    )(x, Win0, Wout0, Win1, Wout1, Win2, Wout2)


# device time: 72255 ns/iter; 1.4693x vs baseline; 1.4693x over previous
import jax
import jax.numpy as jnp
from jax import lax
from jax.experimental import pallas as pl
from jax.experimental.pallas import tpu as pltpu

N_DEV = 4
N_LAYERS = 3

_AG_A = 0
_AG_B = 1
def _ar(l, s):
    return 2 + 3 * l + s


def kernel(x, Win0, Wout0, Win1, Wout1, Win2, Wout2):
    m_per, d = x.shape
    M = N_DEV * m_per
    half = M // 2
    n_ex = 2 + 3 * N_LAYERS

    def body(x_ref, win0_ref, wout0_ref, win1_ref, wout1_ref, win2_ref,
             wout2_ref, out_ref, X_ref, acc_ref, s1_ref, r1_ref,
             r2_ref, send_sems, recv_sems):
        me = lax.axis_index("i")
        p1 = me ^ 1
        p2 = 3 - me

        kh0 = jnp.where((me == 0) | (me == 3), 1, 0)

        wins = (win0_ref, win1_ref, win2_ref)
        wouts = (wout0_ref, wout1_ref, wout2_ref)

        def exchange(idx, src, dst, partner):
            rdma = pltpu.make_async_remote_copy(
                src_ref=src, dst_ref=dst,
                send_sem=send_sems.at[idx], recv_sem=recv_sems.at[idx],
                device_id=(partner,), device_id_type=pl.DeviceIdType.MESH,
            )
            rdma.start()
            return rdma

        def half_partial(l, off):
            xh = X_ref[pl.ds(off, half), :]
            wi = wins[l][...].astype(jnp.bfloat16)
            h = jnp.maximum(
                jnp.dot(xh, wi, preferred_element_type=jnp.float32),
                0.0).astype(jnp.bfloat16)
            wo = wouts[l][...].astype(jnp.bfloat16)
            return jnp.dot(h, wo, preferred_element_type=jnp.float32)

        barrier = pltpu.get_barrier_semaphore()
        for nbr in (p1, p2):
            pl.semaphore_signal(
                barrier, inc=1,
                device_id=(nbr,), device_id_type=pl.DeviceIdType.MESH,
            )
        pl.semaphore_wait(barrier, 2)

        X_ref[pl.ds(me * m_per, m_per), :] = x_ref[...].astype(jnp.bfloat16)
        my_sl = pl.ds(me * m_per, m_per)
        ag_a = exchange(_AG_A, X_ref.at[my_sl, :], X_ref.at[my_sl, :], p1)
        ag_a.wait()
        eh_sl = pl.ds((me // 2) * half, half)
        ag_b = exchange(_AG_B, X_ref.at[eh_sl, :], X_ref.at[eh_sl, :], p2)
        ag_b.wait()
        s1_ref[0] = half_partial(0, (1 - kh0) * half).astype(jnp.bfloat16)

        for l in range(N_LAYERS):
            kh = kh0 ^ (l & 1)
            koff = kh * half
            ksl = pl.ds(koff, half)

            ar1 = exchange(_ar(l, 0), s1_ref.at[l], r1_ref.at[l], p1)
            acc_ref[ksl, :] = half_partial(l, koff)
            ar1.wait()
            acc_ref[ksl, :] += r1_ref[l].astype(jnp.float32)
            X_ref[ksl, :] = acc_ref[ksl, :].astype(jnp.bfloat16)

            ar2 = exchange(_ar(l, 1), X_ref.at[ksl, :], r2_ref.at[l], p2)
            ar2.wait()
            acc_ref[ksl, :] += r2_ref[l].astype(jnp.float32)
            X_ref[ksl, :] = acc_ref[ksl, :].astype(jnp.bfloat16)

            ar3 = exchange(_ar(l, 2), X_ref.at[ksl, :], X_ref.at[ksl, :], p1)
            if l < N_LAYERS - 1:
                s1_ref[l + 1] = half_partial(l + 1, koff).astype(jnp.bfloat16)
            else:
                out_ref[ksl, :] = acc_ref[ksl, :]
            ar3.wait()

        kh_last = kh0 ^ ((N_LAYERS - 1) & 1)
        osl = pl.ds((1 - kh_last) * half, half)
        out_ref[osl, :] = X_ref[osl, :].astype(jnp.float32)

    return pl.pallas_call(
        body,
        out_shape=jax.ShapeDtypeStruct((M, d), jnp.float32),
        in_specs=[pl.BlockSpec(memory_space=pltpu.VMEM)] * 7,
        out_specs=pl.BlockSpec(memory_space=pltpu.VMEM),
        scratch_shapes=[
            pltpu.VMEM((M, d), jnp.bfloat16),
            pltpu.VMEM((M, d), jnp.float32),
            pltpu.VMEM((N_LAYERS, half, d), jnp.bfloat16),
            pltpu.VMEM((N_LAYERS, half, d), jnp.bfloat16),
            pltpu.VMEM((N_LAYERS, half, d), jnp.bfloat16),
            pltpu.SemaphoreType.DMA((n_ex,)),
            pltpu.SemaphoreType.DMA((n_ex,)),
        ],
        compiler_params=pltpu.CompilerParams(
            collective_id=0, vmem_limit_bytes=100 * 1024 * 1024),
    )(x, Win0, Wout0, Win1, Wout1, Win2, Wout2)


# device time: 57604 ns/iter; 1.8430x vs baseline; 1.2543x over previous
import jax
import jax.numpy as jnp
from jax import lax
from jax.experimental import pallas as pl
from jax.experimental.pallas import tpu as pltpu

N_DEV = 4
N_LAYERS = 3

_AG_A = 0
_AG_B = 1
def _ar(l, s):
    return 2 + 3 * l + s


def kernel(x, Win0, Wout0, Win1, Wout1, Win2, Wout2):
    m_per, d = x.shape
    M = N_DEV * m_per
    half = M // 2
    n_ex = 2 + 3 * N_LAYERS

    def body(x_ref, win0_ref, wout0_ref, win1_ref, wout1_ref, win2_ref,
             wout2_ref, out_ref, X_ref, acc_ref, s1_ref, r1_ref, r2_ref,
             wi32_ref, wo32_ref, wi_ref, wo_ref, wdma_sems,
             send_sems, recv_sems):
        me = lax.axis_index("i")
        p1 = me ^ 1
        p2 = 3 - me

        kh0 = jnp.where((me == 0) | (me == 3), 1, 0)

        wins = (win0_ref, win1_ref, win2_ref)
        wouts = (wout0_ref, wout1_ref, wout2_ref)

        def exchange(idx, src, dst, partner):
            rdma = pltpu.make_async_remote_copy(
                src_ref=src, dst_ref=dst,
                send_sem=send_sems.at[idx], recv_sem=recv_sems.at[idx],
                device_id=(partner,), device_id_type=pl.DeviceIdType.MESH,
            )
            rdma.start()
            return rdma

        def fetch_weights(l):
            wi = pltpu.make_async_copy(wins[l], wi32_ref, wdma_sems.at[0])
            wo = pltpu.make_async_copy(wouts[l], wo32_ref, wdma_sems.at[1])
            wi.start()
            wo.start()
            return wi, wo

        def land_weights(dmas):
            for dma in dmas:
                dma.wait()
            wi_ref[...] = wi32_ref[...].astype(jnp.bfloat16)
            wo_ref[...] = wo32_ref[...].astype(jnp.bfloat16)

        def half_partial(off):
            xh = X_ref[pl.ds(off, half), :]
            h = jnp.maximum(
                jnp.dot(xh, wi_ref[...], preferred_element_type=jnp.float32),
                0.0).astype(jnp.bfloat16)
            return jnp.dot(h, wo_ref[...], preferred_element_type=jnp.float32)

        barrier = pltpu.get_barrier_semaphore()
        for nbr in (p1, p2):
            pl.semaphore_signal(
                barrier, inc=1,
                device_id=(nbr,), device_id_type=pl.DeviceIdType.MESH,
            )
        pl.semaphore_wait(barrier, 2)

        X_ref[pl.ds(me * m_per, m_per), :] = x_ref[...].astype(jnp.bfloat16)
        dmas = fetch_weights(0)
        my_sl = pl.ds(me * m_per, m_per)
        ag_a = exchange(_AG_A, X_ref.at[my_sl, :], X_ref.at[my_sl, :], p1)
        ag_a.wait()
        eh_sl = pl.ds((me // 2) * half, half)
        ag_b = exchange(_AG_B, X_ref.at[eh_sl, :], X_ref.at[eh_sl, :], p2)
        land_weights(dmas)
        ag_b.wait()
        s1_ref[0] = half_partial((1 - kh0) * half).astype(jnp.bfloat16)

        for l in range(N_LAYERS):
            kh = kh0 ^ (l & 1)
            koff = kh * half
            ksl = pl.ds(koff, half)

            ar1 = exchange(_ar(l, 0), s1_ref.at[l], r1_ref.at[l], p1)
            acc_ref[ksl, :] = half_partial(koff)
            if l < N_LAYERS - 1:
                dmas = fetch_weights(l + 1)
            ar1.wait()
            acc_ref[ksl, :] += r1_ref[l].astype(jnp.float32)
            X_ref[ksl, :] = acc_ref[ksl, :].astype(jnp.bfloat16)

            ar2 = exchange(_ar(l, 1), X_ref.at[ksl, :], r2_ref.at[l], p2)
            if l < N_LAYERS - 1:
                land_weights(dmas)
            ar2.wait()
            acc_ref[ksl, :] += r2_ref[l].astype(jnp.float32)
            X_ref[ksl, :] = acc_ref[ksl, :].astype(jnp.bfloat16)

            ar3 = exchange(_ar(l, 2), X_ref.at[ksl, :], X_ref.at[ksl, :], p1)
            if l < N_LAYERS - 1:
                s1_ref[l + 1] = half_partial(koff).astype(jnp.bfloat16)
            else:
                out_ref[ksl, :] = acc_ref[ksl, :]
            ar3.wait()

        kh_last = kh0 ^ ((N_LAYERS - 1) & 1)
        osl = pl.ds((1 - kh_last) * half, half)
        out_ref[osl, :] = X_ref[osl, :].astype(jnp.float32)

    return pl.pallas_call(
        body,
        out_shape=jax.ShapeDtypeStruct((M, d), jnp.float32),
        in_specs=[pl.BlockSpec(memory_space=pltpu.VMEM)]
        + [pl.BlockSpec(memory_space=pl.ANY)] * 6,
        out_specs=pl.BlockSpec(memory_space=pltpu.VMEM),
        scratch_shapes=[
            pltpu.VMEM((M, d), jnp.bfloat16),
            pltpu.VMEM((M, d), jnp.float32),
            pltpu.VMEM((N_LAYERS, half, d), jnp.bfloat16),
            pltpu.VMEM((N_LAYERS, half, d), jnp.bfloat16),
            pltpu.VMEM((N_LAYERS, half, d), jnp.bfloat16),
            pltpu.VMEM(Win0.shape, jnp.float32),
            pltpu.VMEM(Wout0.shape, jnp.float32),
            pltpu.VMEM(Win0.shape, jnp.bfloat16),
            pltpu.VMEM(Wout0.shape, jnp.bfloat16),
            pltpu.SemaphoreType.DMA((2,)),
            pltpu.SemaphoreType.DMA((n_ex,)),
            pltpu.SemaphoreType.DMA((n_ex,)),
        ],
        compiler_params=pltpu.CompilerParams(
            collective_id=0, vmem_limit_bytes=100 * 1024 * 1024),
    )(x, Win0, Wout0, Win1, Wout1, Win2, Wout2)


# device time: 52072 ns/iter; 2.0388x vs baseline; 1.1062x over previous
import jax
import jax.numpy as jnp
from jax import lax
from jax.experimental import pallas as pl
from jax.experimental.pallas import tpu as pltpu

N_DEV = 4
N_LAYERS = 3

_AG_A = 0
_AG_B = 1
def _ar(l, s, q):
    return 2 + 6 * l + 2 * s + q


def kernel(x, Win0, Wout0, Win1, Wout1, Win2, Wout2):
    m_per, d = x.shape
    M = N_DEV * m_per
    half = M // 2
    quart = half // 2
    n_ex = 2 + 6 * N_LAYERS

    def body(x_ref, win0_ref, wout0_ref, win1_ref, wout1_ref, win2_ref,
             wout2_ref, out_ref, X_ref, acc_ref, s1_ref, r1_ref, r2_ref,
             wi32_ref, wo32_ref, wi_ref, wo_ref, wdma_sems,
             send_sems, recv_sems):
        me = lax.axis_index("i")
        p1 = me ^ 1
        p2 = 3 - me

        kh0 = jnp.where((me == 0) | (me == 3), 1, 0)

        wins = (win0_ref, win1_ref, win2_ref)
        wouts = (wout0_ref, wout1_ref, wout2_ref)

        def exchange(idx, src, dst, partner):
            rdma = pltpu.make_async_remote_copy(
                src_ref=src, dst_ref=dst,
                send_sem=send_sems.at[idx], recv_sem=recv_sems.at[idx],
                device_id=(partner,), device_id_type=pl.DeviceIdType.MESH,
            )
            rdma.start()
            return rdma

        def fetch_weights(l):
            wi = pltpu.make_async_copy(wins[l], wi32_ref, wdma_sems.at[0])
            wo = pltpu.make_async_copy(wouts[l], wo32_ref, wdma_sems.at[1])
            wi.start()
            wo.start()
            return wi, wo

        def land_weights(dmas):
            for dma in dmas:
                dma.wait()
            wi_ref[...] = wi32_ref[...].astype(jnp.bfloat16)
            wo_ref[...] = wo32_ref[...].astype(jnp.bfloat16)

        def qpart(off):
            xq = X_ref[pl.ds(off, quart), :]
            h = jnp.maximum(
                jnp.dot(xq, wi_ref[...], preferred_element_type=jnp.float32),
                0.0).astype(jnp.bfloat16)
            return jnp.dot(h, wo_ref[...], preferred_element_type=jnp.float32)

        barrier = pltpu.get_barrier_semaphore()
        for nbr in (p1, p2):
            pl.semaphore_signal(
                barrier, inc=1,
                device_id=(nbr,), device_id_type=pl.DeviceIdType.MESH,
            )
        pl.semaphore_wait(barrier, 2)

        X_ref[pl.ds(me * m_per, m_per), :] = x_ref[...].astype(jnp.bfloat16)
        dmas = fetch_weights(0)
        my_sl = pl.ds(me * m_per, m_per)
        ag_a = exchange(_AG_A, X_ref.at[my_sl, :], X_ref.at[my_sl, :], p1)
        ag_a.wait()
        eh_sl = pl.ds((me // 2) * half, half)
        ag_b = exchange(_AG_B, X_ref.at[eh_sl, :], X_ref.at[eh_sl, :], p2)
        land_weights(dmas)
        ag_b.wait()
        soff0 = (1 - kh0) * half
        s1_ref[0, 0] = qpart(soff0).astype(jnp.bfloat16)
        s1_ref[0, 1] = qpart(soff0 + quart).astype(jnp.bfloat16)

        prev_ar3 = None
        for l in range(N_LAYERS):
            kh = kh0 ^ (l & 1)
            koff = kh * half
            q_sl = (pl.ds(koff, quart), pl.ds(koff + quart, quart))

            a1 = [exchange(_ar(l, 0, q), s1_ref.at[l, q], r1_ref.at[l, q], p1)
                  for q in range(2)]
            if prev_ar3 is not None:
                for rdma in prev_ar3:
                    rdma.wait()
            acc_ref[q_sl[0], :] = qpart(koff)
            acc_ref[q_sl[1], :] = qpart(koff + quart)
            if l < N_LAYERS - 1:
                dmas = fetch_weights(l + 1)

            a2 = []
            for q in range(2):
                a1[q].wait()
                acc_ref[q_sl[q], :] += r1_ref[l, q].astype(jnp.float32)
                X_ref[q_sl[q], :] = acc_ref[q_sl[q], :].astype(jnp.bfloat16)
                a2.append(exchange(
                    _ar(l, 1, q), X_ref.at[q_sl[q], :], r2_ref.at[l, q], p2))
            if l < N_LAYERS - 1:
                land_weights(dmas)
            a3 = []
            for q in range(2):
                a2[q].wait()
                if l < N_LAYERS - 1:
                    X_ref[q_sl[q], :] = (
                        acc_ref[q_sl[q], :] + r2_ref[l, q].astype(jnp.float32)
                    ).astype(jnp.bfloat16)
                else:
                    acc_ref[q_sl[q], :] += r2_ref[l, q].astype(jnp.float32)
                    X_ref[q_sl[q], :] = acc_ref[q_sl[q], :].astype(jnp.bfloat16)
                a3.append(exchange(
                    _ar(l, 2, q), X_ref.at[q_sl[q], :], X_ref.at[q_sl[q], :],
                    p1))
            if l < N_LAYERS - 1:
                s1_ref[l + 1, 0] = qpart(koff).astype(jnp.bfloat16)
                s1_ref[l + 1, 1] = qpart(koff + quart).astype(jnp.bfloat16)
                prev_ar3 = a3
            else:
                out_ref[pl.ds(koff, half), :] = acc_ref[pl.ds(koff, half), :]
                for rdma in a3:
                    rdma.wait()
                osl = pl.ds((1 - kh) * half, half)
                out_ref[osl, :] = X_ref[osl, :].astype(jnp.float32)

    return pl.pallas_call(
        body,
        out_shape=jax.ShapeDtypeStruct((M, d), jnp.float32),
        in_specs=[pl.BlockSpec(memory_space=pltpu.VMEM)]
        + [pl.BlockSpec(memory_space=pl.ANY)] * 6,
        out_specs=pl.BlockSpec(memory_space=pltpu.VMEM),
        scratch_shapes=[
            pltpu.VMEM((M, d), jnp.bfloat16),
            pltpu.VMEM((M, d), jnp.float32),
            pltpu.VMEM((N_LAYERS, 2, quart, d), jnp.bfloat16),
            pltpu.VMEM((N_LAYERS, 2, quart, d), jnp.bfloat16),
            pltpu.VMEM((N_LAYERS, 2, quart, d), jnp.bfloat16),
            pltpu.VMEM(Win0.shape, jnp.float32),
            pltpu.VMEM(Wout0.shape, jnp.float32),
            pltpu.VMEM(Win0.shape, jnp.bfloat16),
            pltpu.VMEM(Wout0.shape, jnp.bfloat16),
            pltpu.SemaphoreType.DMA((2,)),
            pltpu.SemaphoreType.DMA((n_ex,)),
            pltpu.SemaphoreType.DMA((n_ex,)),
        ],
        compiler_params=pltpu.CompilerParams(
            collective_id=0, vmem_limit_bytes=100 * 1024 * 1024),
    )(x, Win0, Wout0, Win1, Wout1, Win2, Wout2)
